# baseline (speedup 1.0000x reference)
import functools

import jax
import jax.numpy as jnp
from jax import lax
from jax.experimental import pallas as pl
from jax.experimental.pallas import tpu as pltpu
from jax.experimental.pallas import tpu_sc as plsc

NSITES = 16384
D = 128
FSC = 16
FTC = D - FSC
PACK = 128 // FSC
NC = 2
NS = 16
NW = NC * NS
CPW = NSITES // NW
S = 128
NSUB = CPW // S
L = 16
PT = S + 1
EP = 4 * FSC + 1
BN = 2048

_mesh = plsc.VectorSubcoreMesh(core_axis_name="c", subcore_axis_name="s")


@functools.partial(
    pl.kernel,
    out_type=jax.ShapeDtypeStruct((FSC, NSITES), jnp.float32),
    mesh=_mesh,
    compiler_params=pltpu.CompilerParams(needs_layout_passes=False),
    scratch_types=[
        pltpu.VMEM((2, S), jnp.float32),
        pltpu.VMEM((2, S), jnp.float32),
        pltpu.VMEM((S // PACK, S), jnp.float32),
        pltpu.VMEM((S // PACK, S), jnp.float32),
        pltpu.VMEM((FSC, PT), jnp.float32),
        pltpu.VMEM((FSC, PT), jnp.float32),
        pltpu.VMEM((L * EP,), jnp.float32),
        pltpu.SemaphoreType.DMA,
        pltpu.SemaphoreType.DMA,
        pltpu.SemaphoreType.DMA,
        pltpu.SemaphoreType.DMA,
    ],
)
def _sc_body(x_hbm, e_hbm, p_hbm, out_hbm,
             x_v0, x_v1, p_v0, p_v1, pt_v0, pt_v1, e_v,
             in_s0, in_s1, out_s0, out_s1):
    cid = lax.axis_index("c")
    sid = lax.axis_index("s")
    wid = sid * NC + cid
    base = wid * CPW
    pltpu.sync_copy(e_hbm, e_v)

    xv = (x_v0, x_v1)
    pv = (p_v0, p_v1)
    ptv = (pt_v0, pt_v1)
    ins = (in_s0, in_s1)
    outs = (out_s0, out_s1)

    iota = lax.iota(jnp.int32, L)
    lane_base = iota * jnp.full((L,), EP, jnp.int32)
    zf = jnp.zeros((L,), jnp.float32)
    zi = jnp.zeros((L,), jnp.int32)
    onei = jnp.ones((L,), jnp.int32)
    twoi = jnp.full((L,), 2, jnp.int32)

    def start_in(k):
        b = k % 2
        sb = base + k * S
        d1 = pltpu.async_copy(x_hbm.at[:, pl.ds(sb, S)], xv[b], ins[b])
        d2 = pltpu.async_copy(
            p_hbm.at[pl.ds(pl.multiple_of(sb // PACK, S // PACK), S // PACK),
                     :], pv[b], ins[b])
        return (d1, d2)

    pending_in = {0: start_in(0), 1: None}
    pending_out = {0: None, 1: None}
    for k in range(NSUB):
        b = k % 2
        if k + 1 < NSUB:
            pending_in[1 - b] = start_in(k + 1)
        for dsc in pending_in[b]:
            dsc.wait()
        if pending_out[b] is not None:
            pending_out[b].wait()
        for s in range(S):
            g = pv[b][s // PACK, pl.ds((s % PACK) * FSC, FSC)]
            plsc.store_scatter(
                ptv[b], [iota, jnp.full((L,), s, jnp.int32)], g)
        for j0 in range(0, S, L):
            v0 = xv[b][0, pl.ds(j0, L)]
            v1 = xv[b][1, pl.ds(j0, L)]
            idx16 = jnp.where(v0 < zf, twoi, zi) + jnp.where(v1 < zf, onei, zi)
            e_base = lane_base + idx16
            cols = iota + jnp.full((L,), j0, jnp.int32)
            for d in range(FSC):
                e = plsc.load_gather(e_v, [e_base + 4 * d])
                r = ptv[b][d, pl.ds(j0, L)]
                plsc.store_scatter(
                    ptv[b], [jnp.full((L,), d, jnp.int32), cols], r + e)
        sb = base + k * S
        pending_out[b] = pltpu.async_copy(
            ptv[b].at[:, pl.ds(0, S)], out_hbm.at[:, pl.ds(sb, S)], outs[b])
    for b in range(2):
        if pending_out[b] is not None:
            pending_out[b].wait()


def _tc_body(x_ref, et_ref, p_ref, o_ref):
    pt = p_ref[:].T[FSC:, :]
    b0 = x_ref[0:1, :] < 0.0
    b1 = x_ref[1:2, :] < 0.0
    et = et_ref[:]
    e0 = et[:, 0:1]
    e1 = et[:, 1:2]
    e2 = et[:, 2:3]
    e3 = et[:, 3:4]
    sel = jnp.where(b0, jnp.where(b1, e3, e2), jnp.where(b1, e1, e0))
    o_ref[:] = pt + sel


def kernel(x, E, P):
    et = E.T
    psub = P[:, :FSC].reshape(NSITES // PACK, PACK * FSC)
    esub = jnp.pad(et[:FSC, :].reshape(-1), (0, EP - 4 * FSC))
    ef = jnp.tile(esub, (L,))
    sc_out = _sc_body(x, ef, psub)

    tc_out = pl.pallas_call(
        _tc_body,
        grid=(NSITES // BN,),
        in_specs=[
            pl.BlockSpec((2, BN), lambda i: (0, i)),
            pl.BlockSpec((FTC, 4), lambda i: (0, 0)),
            pl.BlockSpec((BN, D), lambda i: (i, 0)),
        ],
        out_specs=pl.BlockSpec((FTC, BN), lambda i: (0, i)),
        out_shape=jax.ShapeDtypeStruct((FTC, NSITES), jnp.float32),
    )(x, et[FSC:], P)

    return jnp.concatenate([sc_out, tc_out], axis=0)
